# baseline (speedup 1.0000x reference)
import functools

import jax
import jax.numpy as jnp
from jax import lax
from jax.experimental import pallas as pl
from jax.experimental.pallas import tpu as pltpu
from jax.experimental.pallas import tpu_sc as plsc

B = 4096
CTX = 20
E = 64
SUB = 2
VOCAB_BLOCKS = 1000000 // SUB

NC = 2
NS = 16
NW = NC * NS
BPW = B // NW
NBG = BPW // 16
TWIN = (8, 8, 4)
TOFF = (0, 8, 16)
MAXI = 128


def _sc_gather_body(table_hbm, ictx_hbm, o_hbm, in_sumT_hbm, out_embT_hbm,
                    idx_v, sub_v, oidx_v, otidx_v, blocks_v, accT_v, sem):
  wid = lax.axis_index("s") * NC + lax.axis_index("c")
  base = wid * BPW
  lanes = lax.iota(jnp.int32, 16)

  pltpu.sync_copy(ictx_hbm.at[wid], idx_v)

  @pl.loop(0, NBG)
  def _per_group(bg):
    for h, th in enumerate(TWIN):
      off = TOFF[h] * 16
      for t in range(th):
        v = idx_v[bg, pl.ds(off + t * 16, 16)]
        sub_v[t, :] = (v & 1) * E
        idx_v[bg, pl.ds(off + t * 16, 16)] = lax.shift_right_logical(v, 1)
      pltpu.async_copy(
          table_hbm.at[idx_v.at[bg].at[pl.ds(off, th * 16)]],
          blocks_v.at[pl.ds(0, th * 16)], sem).wait()

      @pl.loop(0, E)
      def _per_col(c):
        cvec = jnp.full((16,), c, dtype=jnp.int32)
        if h == 0:
          acc = plsc.load_gather(blocks_v, [lanes, sub_v[0, :] + cvec])
          t0 = 1
        else:
          acc = accT_v[c, pl.ds(bg * 16, 16)]
          t0 = 0
        for t in range(t0, th):
          acc = acc + plsc.load_gather(
              blocks_v, [t * 16 + lanes, sub_v[t, :] + cvec])
        accT_v[c, pl.ds(bg * 16, 16)] = acc

  pltpu.sync_copy(accT_v, in_sumT_hbm.at[:, pl.ds(base, BPW)])

  pltpu.sync_copy(o_hbm.at[pl.ds(base, BPW)], oidx_v)
  for g in range(NBG):
    v = oidx_v[pl.ds(g * 16, 16)]
    sub_v[g, :] = (v & 1) * E
    otidx_v[pl.ds(g * 16, 16)] = lax.shift_right_logical(v, 1)
  pltpu.async_copy(
      table_hbm.at[otidx_v],
      blocks_v.at[pl.ds(0, BPW)], sem).wait()

  @pl.loop(0, E)
  def _per_col_o(c):
    cvec = jnp.full((16,), c, dtype=jnp.int32)
    for g in range(NBG):
      val = plsc.load_gather(blocks_v, [g * 16 + lanes, sub_v[g, :] + cvec])
      accT_v[c, pl.ds(g * 16, 16)] = val

  pltpu.sync_copy(accT_v, out_embT_hbm.at[:, pl.ds(base, BPW)])


def _sc_gather(table3, ictx, o):
  mesh = plsc.VectorSubcoreMesh(core_axis_name="c", subcore_axis_name="s")
  f = pl.kernel(
      _sc_gather_body,
      out_type=(
          jax.ShapeDtypeStruct((E, B), jnp.float32),
          jax.ShapeDtypeStruct((E, B), jnp.float32),
      ),
      mesh=mesh,
      compiler_params=pltpu.CompilerParams(needs_layout_passes=False),
      scratch_types=[
          pltpu.VMEM((NBG, CTX * 16), jnp.int32),
          pltpu.VMEM((max(TWIN), 16), jnp.int32),
          pltpu.VMEM((BPW,), jnp.int32),
          pltpu.VMEM((BPW,), jnp.int32),
          pltpu.VMEM((MAXI, SUB * E), jnp.float32),
          pltpu.VMEM((E, BPW), jnp.float32),
          pltpu.SemaphoreType.DMA,
      ],
  )
  return f(table3, ictx, o)


def _tc_score_body(a_ref, b_ref, o_ref):
  a = a_ref[...] * (1.0 / CTX)
  s = lax.dot_general(
      a, b_ref[...], (((0,), (0,)), ((), ())),
      preferred_element_type=jnp.float32,
      precision=lax.Precision.DEFAULT,
  )
  o_ref[...] = jnp.minimum(s, 0.0) - jnp.log1p(jnp.exp(-jnp.abs(s)))


def _tc_score(in_sumT, out_embT):
  BM = 512
  grid = (B // BM,)
  return pl.pallas_call(
      _tc_score_body,
      grid=grid,
      in_specs=[
          pl.BlockSpec((E, BM), lambda m: (0, m)),
          pl.BlockSpec((E, B), lambda m: (0, 0)),
      ],
      out_specs=pl.BlockSpec((BM, B), lambda m: (m, 0)),
      out_shape=jax.ShapeDtypeStruct((B, B), jnp.float32),
  )(in_sumT, out_embT)


@jax.jit
def kernel(i, o, table):
  table3 = table.reshape(VOCAB_BLOCKS, SUB * E)
  ictx = (i.reshape(NW, NBG, 16, CTX)
           .transpose(0, 1, 3, 2)
           .reshape(NW, NBG, CTX * 16))
  in_sumT, out_embT = _sc_gather(table3, ictx, o)
  return _tc_score(in_sumT, out_embT)


# speedup vs baseline: 1.1779x; 1.1779x over previous
import functools

import jax
import jax.numpy as jnp
from jax import lax
from jax.experimental import pallas as pl
from jax.experimental.pallas import tpu as pltpu
from jax.experimental.pallas import tpu_sc as plsc

B = 4096
CTX = 20
E = 64

NC = 2
NS = 16
NW = NC * NS
BPW = B // NW
NCHUNK = 4
CPB = BPW // NCHUNK
ROWS_PER_CHUNK = CPB * CTX


def _sc_gather_body(table_hbm, ictx_hbm, o_hbm, in_sum_hbm, out_emb_hbm,
                    idx_v, oidx_v, rows_v, acc_v, orow_v, sem, osem):
  wid = lax.axis_index("s") * NC + lax.axis_index("c")
  base = wid * BPW

  pltpu.sync_copy(ictx_hbm.at[wid], idx_v)
  pltpu.sync_copy(o_hbm.at[pl.ds(base, BPW)], oidx_v)

  ocopy = pltpu.async_copy(table_hbm.at[oidx_v], orow_v, osem)

  for c in range(NCHUNK):
    pltpu.async_copy(table_hbm.at[idx_v.at[c]], rows_v, sem).wait()

    @pl.loop(0, CPB)
    def _sum_rows(b):
      row0 = b * CTX
      for k in range(E // 16):
        cols = pl.ds(k * 16, 16)
        acc = rows_v[row0, cols]
        for t in range(1, CTX):
          acc = acc + rows_v[row0 + t, cols]
        acc_v[c * CPB + b, cols] = acc

  pltpu.sync_copy(acc_v, in_sum_hbm.at[pl.ds(base, BPW)])
  ocopy.wait()
  pltpu.sync_copy(orow_v, out_emb_hbm.at[pl.ds(base, BPW)])


def _sc_gather(table, ictx, o):
  mesh = plsc.VectorSubcoreMesh(core_axis_name="c", subcore_axis_name="s")
  f = pl.kernel(
      _sc_gather_body,
      out_type=(
          jax.ShapeDtypeStruct((B, E), jnp.float32),
          jax.ShapeDtypeStruct((B, E), jnp.float32),
      ),
      mesh=mesh,
      compiler_params=pltpu.CompilerParams(use_tc_tiling_on_sc=False),
      scratch_types=[
          pltpu.VMEM((NCHUNK, ROWS_PER_CHUNK), jnp.int32),
          pltpu.VMEM((BPW,), jnp.int32),
          pltpu.VMEM((ROWS_PER_CHUNK, E), jnp.float32),
          pltpu.VMEM((BPW, E), jnp.float32),
          pltpu.VMEM((BPW, E), jnp.float32),
          pltpu.SemaphoreType.DMA,
          pltpu.SemaphoreType.DMA,
      ],
  )
  return f(table, ictx, o)


def _tc_score_body(a_ref, b_ref, o_ref):
  a = a_ref[...] * (1.0 / CTX)
  s = lax.dot_general(
      a, b_ref[...], (((1,), (1,)), ((), ())),
      preferred_element_type=jnp.float32,
      precision=lax.Precision.DEFAULT,
  )
  o_ref[...] = jnp.minimum(s, 0.0) - jnp.log1p(jnp.exp(-jnp.abs(s)))


def _tc_score(in_sum, out_emb):
  BM = 512
  grid = (B // BM,)
  return pl.pallas_call(
      _tc_score_body,
      grid=grid,
      in_specs=[
          pl.BlockSpec((BM, E), lambda m: (m, 0)),
          pl.BlockSpec((B, E), lambda m: (0, 0)),
      ],
      out_specs=pl.BlockSpec((BM, B), lambda m: (m, 0)),
      out_shape=jax.ShapeDtypeStruct((B, B), jnp.float32),
  )(in_sum, out_emb)


@jax.jit
def kernel(i, o, table):
  ictx = i.reshape(NW, NCHUNK, ROWS_PER_CHUNK)
  in_sum, out_emb = _sc_gather(table, ictx, o)
  return _tc_score(in_sum, out_emb)
